# Initial kernel scaffold; baseline (speedup 1.0000x reference)
import functools

import jax
import jax.numpy as jnp
from jax import lax
from jax.experimental import pallas as pl
from jax.experimental.pallas import tpu as pltpu
from jax.experimental.pallas import tpu_sc as plsc

N = 10000
E = 320000
D = 128

NC = 2
NS = 16
CHUNK = 128
Dh = D // NC
EP = ((E + NS * CHUNK - 1) // (NS * CHUNK)) * (NS * CHUNK)
NCH = EP // (NS * CHUNK)
STRIPE = N // NS
NPAD = N + 16


def _sc_body(x0_hbm, x1_hbm, row_hbm, col_hbm,
             agg0_hbm, agg1_hbm, deg_hbm,
             xs_s, agg_s, deg_s,
             row_v, col_v, gbuf, wbuf, dbuf, ones_v):
    c = lax.axis_index("c")
    s = lax.axis_index("s")
    zeros16 = jnp.zeros((16,), jnp.float32)
    ones16 = jnp.ones((16,), jnp.float32)

    def zrow(r, _):
        def zcol(k, _):
            wbuf[r, pl.ds(k * 16, 16)] = zeros16
            return 0
        return lax.fori_loop(0, Dh // 16, zcol, 0)
    lax.fori_loop(0, STRIPE, zrow, 0)

    def zdeg(i, _):
        dbuf[pl.ds(i * 16, 16)] = zeros16
        return 0
    lax.fori_loop(0, NPAD // 16, zdeg, 0)

    def fones(i, _):
        ones_v[pl.ds(i * 16, 16)] = ones16
        return 0
    lax.fori_loop(0, CHUNK // 16, fones, 0)

    pltpu.sync_copy(wbuf, agg_s.at[pl.ds(s * STRIPE, STRIPE)])

    @pl.when(s == 0)
    def _():
        pltpu.sync_copy(wbuf.at[pl.ds(0, NPAD - N)], agg_s.at[pl.ds(N, NPAD - N)])
        pltpu.sync_copy(dbuf, deg_s)

    @pl.when(c == 0)
    def _():
        pltpu.sync_copy(x0_hbm.at[pl.ds(s * STRIPE, STRIPE)], wbuf)

    @pl.when(c == 1)
    def _():
        pltpu.sync_copy(x1_hbm.at[pl.ds(s * STRIPE, STRIPE)], wbuf)

    pltpu.sync_copy(wbuf, xs_s.at[pl.ds(s * STRIPE, STRIPE)])

    pltpu.sync_copy(row_hbm.at[s], row_v)
    pltpu.sync_copy(col_hbm.at[s], col_v)

    plsc.subcore_barrier()

    def body_j(j, _):
        pltpu.sync_copy(xs_s.at[col_v.at[j]], gbuf)
        pltpu.sync_copy(gbuf, agg_s.at[row_v.at[j]], add=True)
        return 0
    lax.fori_loop(0, NCH, body_j, 0)

    @pl.when(c == 0)
    def _():
        def dloop(j, _):
            pltpu.sync_copy(ones_v, deg_s.at[row_v.at[j]], add=True)
            return 0
        lax.fori_loop(0, NCH, dloop, 0)

    plsc.subcore_barrier()

    pltpu.sync_copy(agg_s.at[pl.ds(s * STRIPE, STRIPE)], wbuf)

    @pl.when(c == 0)
    def _():
        pltpu.sync_copy(wbuf, agg0_hbm.at[pl.ds(s * STRIPE, STRIPE)])

    @pl.when(c == 1)
    def _():
        pltpu.sync_copy(wbuf, agg1_hbm.at[pl.ds(s * STRIPE, STRIPE)])

    @pl.when((c == 0) & (s == 0))
    def _():
        pltpu.sync_copy(deg_s.at[pl.ds(0, N)], dbuf.at[pl.ds(0, N)])
        pltpu.sync_copy(dbuf.at[pl.ds(0, N)], deg_hbm)


def _sc_aggregate(x0, x1, row_r, col_r):
    mesh = plsc.VectorSubcoreMesh(
        core_axis_name="c", subcore_axis_name="s", num_cores=NC, num_subcores=NS
    )
    f32 = jnp.float32
    return pl.kernel(
        _sc_body,
        out_type=[
            jax.ShapeDtypeStruct((N, Dh), f32),
            jax.ShapeDtypeStruct((N, Dh), f32),
            jax.ShapeDtypeStruct((N,), f32),
        ],
        mesh=mesh,
        scratch_types=[
            pltpu.VMEM_SHARED((N, Dh), f32),
            pltpu.VMEM_SHARED((NPAD, Dh), f32),
            pltpu.VMEM_SHARED((NPAD,), f32),
            pltpu.VMEM((NCH, CHUNK), jnp.int32),
            pltpu.VMEM((NCH, CHUNK), jnp.int32),
            pltpu.VMEM((CHUNK, Dh), f32),
            pltpu.VMEM((STRIPE, Dh), f32),
            pltpu.VMEM((NPAD,), f32),
            pltpu.VMEM((CHUNK,), f32),
        ],
    )(x0, x1, row_r, col_r)


def _tc_body(a0_ref, a1_ref, deg_ref, w0_ref, w1_ref, b_ref, o_ref):
    deg = deg_ref[...]
    dinv = jnp.where(deg > 0.0, 1.0 / deg, 0.0)
    a0 = a0_ref[...] * dinv
    a1 = a1_ref[...] * dinv
    o_ref[...] = (
        jnp.dot(a0, w0_ref[...], preferred_element_type=jnp.float32)
        + jnp.dot(a1, w1_ref[...], preferred_element_type=jnp.float32)
        + b_ref[...]
    )


def _tc_epilogue(agg0, agg1, deg, W0, W1, b2):
    Bn = 1000
    grid = (N // Bn,)
    return pl.pallas_call(
        _tc_body,
        grid=grid,
        in_specs=[
            pl.BlockSpec((Bn, Dh), lambda i: (i, 0)),
            pl.BlockSpec((Bn, Dh), lambda i: (i, 0)),
            pl.BlockSpec((Bn, 1), lambda i: (i, 0)),
            pl.BlockSpec((Dh, D), lambda i: (0, 0)),
            pl.BlockSpec((Dh, D), lambda i: (0, 0)),
            pl.BlockSpec((1, D), lambda i: (0, 0)),
        ],
        out_specs=pl.BlockSpec((Bn, D), lambda i: (i, 0)),
        out_shape=jax.ShapeDtypeStruct((N, D), jnp.float32),
    )(agg0, agg1, deg, W0, W1, b2)


def kernel(x, edge_index, W, b):
    row = edge_index[0].astype(jnp.int32)
    col = edge_index[1].astype(jnp.int32)
    pad = EP - E
    rowp = jnp.concatenate([row, jnp.full((pad,), N, jnp.int32)])
    colp = jnp.concatenate([col, jnp.zeros((pad,), jnp.int32)])
    row_r = rowp.reshape(NS, NCH, CHUNK)
    col_r = colp.reshape(NS, NCH, CHUNK)
    x0 = x[:, :Dh]
    x1 = x[:, Dh:]

    agg0, agg1, deg = _sc_aggregate(x0, x1, row_r, col_r)

    W0 = W[:, :Dh].T
    W1 = W[:, Dh:].T
    return _tc_epilogue(agg0, agg1, deg.reshape(N, 1), W0, W1, b.reshape(1, D))


# speedup vs baseline: 14.0591x; 14.0591x over previous
import jax
import jax.numpy as jnp
from jax import lax
from jax.experimental import pallas as pl
from jax.experimental.pallas import tpu as pltpu
from jax.experimental.pallas import tpu_sc as plsc

N = 10000
E = 320000
D = 128

NC = 2
NS = 16
CHUNK = 128
IBLK = 8
Dh = D // NC
NCH = 160
EP = NS * NCH * CHUNK
NBLK = NCH // IBLK
NP = 10240
STRIPE = NP // NS


def _sc_body(x0_hbm, x1_hbm, row_hbm, col_hbm,
             agg0_hbm, agg1_hbm, deg_hbm,
             xs_s, agg_s, deg_s,
             rbuf, cbuf, gbuf, zbuf, ones_v):
    c = lax.axis_index("c")
    s = lax.axis_index("s")
    zeros16 = jnp.zeros((16,), jnp.float32)
    ones16 = jnp.ones((16,), jnp.float32)

    def zrow(r, _):
        def zcol(k, _):
            gbuf[r, pl.ds(k * 16, 16)] = zeros16
            return 0
        return lax.fori_loop(0, Dh // 16, zcol, 0)
    lax.fori_loop(0, CHUNK, zrow, 0)

    def z1(i, _):
        zbuf[pl.ds(i * 16, 16)] = zeros16
        ones_v[pl.ds(i * 16, 16)] = ones16
        return 0
    lax.fori_loop(0, CHUNK // 16, z1, 0)

    def zagg(k, _):
        pltpu.sync_copy(gbuf, agg_s.at[pl.ds(s * STRIPE + k * CHUNK, CHUNK)])
        pltpu.sync_copy(zbuf, deg_s.at[pl.ds(s * STRIPE + k * CHUNK, CHUNK)])
        return 0
    lax.fori_loop(0, STRIPE // CHUNK, zagg, 0)

    @pl.when(c == 0)
    def _():
        pltpu.sync_copy(x0_hbm.at[pl.ds(s * STRIPE, STRIPE)],
                        xs_s.at[pl.ds(s * STRIPE, STRIPE)])

    @pl.when(c == 1)
    def _():
        pltpu.sync_copy(x1_hbm.at[pl.ds(s * STRIPE, STRIPE)],
                        xs_s.at[pl.ds(s * STRIPE, STRIPE)])

    plsc.subcore_barrier()

    def blk(t, _):
        pltpu.sync_copy(row_hbm.at[s, pl.ds(t * IBLK, IBLK)], rbuf)
        pltpu.sync_copy(col_hbm.at[s, pl.ds(t * IBLK, IBLK)], cbuf)

        def chunk(j, _):
            pltpu.sync_copy(xs_s.at[cbuf.at[j]], gbuf)
            pltpu.sync_copy(gbuf, agg_s.at[rbuf.at[j]], add=True)

            @pl.when(c == 0)
            def _():
                pltpu.sync_copy(ones_v, deg_s.at[rbuf.at[j]], add=True)
            return 0
        return lax.fori_loop(0, IBLK, chunk, 0)
    lax.fori_loop(0, NBLK, blk, 0)

    plsc.subcore_barrier()

    @pl.when(c == 0)
    def _():
        pltpu.sync_copy(agg_s.at[pl.ds(s * STRIPE, STRIPE)],
                        agg0_hbm.at[pl.ds(s * STRIPE, STRIPE)])

    @pl.when(c == 1)
    def _():
        pltpu.sync_copy(agg_s.at[pl.ds(s * STRIPE, STRIPE)],
                        agg1_hbm.at[pl.ds(s * STRIPE, STRIPE)])

    @pl.when((c == 0) & (s == 0))
    def _():
        pltpu.sync_copy(deg_s.at[pl.ds(0, N)], deg_hbm)


def _sc_aggregate(x0, x1, row_r, col_r):
    mesh = plsc.VectorSubcoreMesh(
        core_axis_name="c", subcore_axis_name="s", num_cores=NC, num_subcores=NS
    )
    f32 = jnp.float32
    return pl.kernel(
        _sc_body,
        out_type=[
            jax.ShapeDtypeStruct((NP, Dh), f32),
            jax.ShapeDtypeStruct((NP, Dh), f32),
            jax.ShapeDtypeStruct((N,), f32),
        ],
        mesh=mesh,
        compiler_params=pltpu.CompilerParams(use_tc_tiling_on_sc=False),
        scratch_types=[
            pltpu.VMEM_SHARED((NP, Dh), f32),
            pltpu.VMEM_SHARED((NP, Dh), f32),
            pltpu.VMEM_SHARED((NP,), f32),
            pltpu.VMEM((IBLK, CHUNK), jnp.int32),
            pltpu.VMEM((IBLK, CHUNK), jnp.int32),
            pltpu.VMEM((CHUNK, Dh), f32),
            pltpu.VMEM((CHUNK,), f32),
            pltpu.VMEM((CHUNK,), f32),
        ],
    )(x0, x1, row_r, col_r)


def _tc_body(a0_ref, a1_ref, deg_ref, w0_ref, w1_ref, b_ref, o_ref):
    deg = deg_ref[...]
    dinv = jnp.where(deg > 0.0, 1.0 / deg, 0.0)
    a0 = a0_ref[...] * dinv
    a1 = a1_ref[...] * dinv
    o_ref[...] = (
        jnp.dot(a0, w0_ref[...], preferred_element_type=jnp.float32)
        + jnp.dot(a1, w1_ref[...], preferred_element_type=jnp.float32)
        + b_ref[...]
    )


def _tc_epilogue(agg0, agg1, deg, W0, W1, b2):
    Bn = 1000
    grid = (N // Bn,)
    return pl.pallas_call(
        _tc_body,
        grid=grid,
        in_specs=[
            pl.BlockSpec((Bn, Dh), lambda i: (i, 0)),
            pl.BlockSpec((Bn, Dh), lambda i: (i, 0)),
            pl.BlockSpec((Bn, 1), lambda i: (i, 0)),
            pl.BlockSpec((Dh, D), lambda i: (0, 0)),
            pl.BlockSpec((Dh, D), lambda i: (0, 0)),
            pl.BlockSpec((1, D), lambda i: (0, 0)),
        ],
        out_specs=pl.BlockSpec((Bn, D), lambda i: (i, 0)),
        out_shape=jax.ShapeDtypeStruct((N, D), jnp.float32),
    )(agg0, agg1, deg, W0, W1, b2)


def kernel(x, edge_index, W, b):
    row = edge_index[0].astype(jnp.int32)
    col = edge_index[1].astype(jnp.int32)
    pad = EP - E
    rowp = jnp.concatenate([row, jnp.full((pad,), N, jnp.int32)])
    colp = jnp.concatenate([col, jnp.zeros((pad,), jnp.int32)])
    row_r = rowp.reshape(NS, NCH, CHUNK)
    col_r = colp.reshape(NS, NCH, CHUNK)
    xp = jnp.concatenate([x, jnp.zeros((NP - N, D), x.dtype)], axis=0)
    x0 = xp[:, :Dh]
    x1 = xp[:, Dh:]

    agg0, agg1, deg = _sc_aggregate(x0, x1, row_r, col_r)

    W0 = W[:, :Dh].T
    W1 = W[:, Dh:].T
    return _tc_epilogue(agg0, agg1, deg.reshape(N, 1), W0, W1, b.reshape(1, D))
